# Initial kernel scaffold; baseline (speedup 1.0000x reference)
import jax
import jax.numpy as jnp
from jax.experimental import pallas as pl


def kernel(x, emb0, emb1, emb2, emb3, W1, W2, W3):
    raise NotImplementedError("write your pallas kernel here")


# speedup vs baseline: 8.0320x; 8.0320x over previous
import functools

import jax
import jax.numpy as jnp
from jax import lax
from jax.experimental import pallas as pl
from jax.experimental.pallas import tpu as pltpu
from jax.experimental.pallas import tpu_sc as plsc

EMB_DIM = 128
_NC, _NS = 2, 16
_NW = _NC * _NS
_CH = 128


def _proj_body(t_ref, w_ref, o_ref):
    o_ref[...] = jnp.dot(t_ref[...], w_ref[...],
                         preferred_element_type=jnp.float32)


def _project(table, w, blk=2048):
    r, k = table.shape
    return pl.pallas_call(
        _proj_body,
        grid=(pl.cdiv(r, blk),),
        in_specs=[pl.BlockSpec((blk, k), lambda i: (i, 0)),
                  pl.BlockSpec((k, EMB_DIM), lambda i: (0, 0))],
        out_specs=pl.BlockSpec((blk, EMB_DIM), lambda i: (i, 0)),
        out_shape=jax.ShapeDtypeStruct((r, EMB_DIM), jnp.float32),
    )(table, w)


def _gidx_body(x_ref, o_ref):
    v = x_ref[...]
    shift = ((v > 10000).astype(jnp.int32)
             + (v > 20000).astype(jnp.int32)
             + (v > 40000).astype(jnp.int32))
    o_ref[...] = jnp.where(v == 0, 10000, v + shift)


def _gidx(x2):
    return pl.pallas_call(
        _gidx_body,
        out_shape=jax.ShapeDtypeStruct(x2.shape, jnp.int32),
    )(x2)


def _sc_gather(gidx3, table):
    nch = gidx3.shape[1]
    n_out = _NW * nch * _CH
    mesh = plsc.VectorSubcoreMesh(core_axis_name="c", subcore_axis_name="s")

    @functools.partial(
        pl.kernel,
        out_type=jax.ShapeDtypeStruct((n_out, EMB_DIM), jnp.float32),
        mesh=mesh,
        scratch_types=[
            pltpu.VMEM((nch, _CH), jnp.int32),
            pltpu.VMEM((2, _CH, EMB_DIM), jnp.float32),
            pltpu.SemaphoreType.DMA,
            pltpu.SemaphoreType.DMA,
        ],
    )
    def k(idx_hbm, table_hbm, out_hbm, idx_v, rows_v, sem0, sem1):
        wid = lax.axis_index("s") * _NC + lax.axis_index("c")
        cbase = wid * nch
        pltpu.sync_copy(idx_hbm.at[wid], idx_v)

        def fire(j, buf, sem):
            pltpu.async_copy(table_hbm.at[idx_v.at[j]], buf, sem)

        def wait(j, buf, sem):
            pltpu.make_async_copy(table_hbm.at[idx_v.at[j]], buf, sem).wait()

        def store(j, buf):
            pltpu.sync_copy(buf, out_hbm.at[pl.ds((cbase + j) * _CH, _CH)])

        fire(0, rows_v.at[0], sem0)

        def body(s, carry):
            j0 = 2 * s
            j1 = j0 + 1
            fire(j1, rows_v.at[1], sem1)
            wait(j0, rows_v.at[0], sem0)
            store(j0, rows_v.at[0])
            j2 = jnp.minimum(j0 + 2, nch - 1)
            fire(j2, rows_v.at[0], sem0)
            wait(j1, rows_v.at[1], sem1)
            store(j1, rows_v.at[1])
            return carry

        lax.fori_loop(0, nch // 2, body, 0)
        wait(nch - 1, rows_v.at[0], sem0)

    return k(gidx3, table)


def kernel(x, emb0, emb1, emb2, emb3, W1, W2, W3):
    x2 = x.reshape(-1, _CH).astype(jnp.int32)
    g2 = _gidx(x2)
    p1 = _project(emb1, W1)
    p2 = _project(emb2, W2)
    p3 = _project(emb3, W3)
    table = jnp.concatenate([emb0, p1, p2, p3], axis=0)
    out = _sc_gather(g2.reshape(_NW, -1, _CH), table)
    return out.reshape(x.shape + (EMB_DIM,))


# speedup vs baseline: 10.6376x; 1.3244x over previous
import functools

import jax
import jax.numpy as jnp
from jax import lax
from jax.experimental import pallas as pl
from jax.experimental.pallas import tpu as pltpu
from jax.experimental.pallas import tpu_sc as plsc

EMB_DIM = 128
_NC, _NS = 2, 16
_NW = _NC * _NS
_CH = 128

_BLK = 2048
_NB = (5, 5, 10, 30)


def _table_body(e0, e1, e2, e3, w1, w2, w3, o):
    i = pl.program_id(0)

    @pl.when(i < 5)
    def _():
        o[...] = e0[...]

    @pl.when((i >= 5) & (i < 10))
    def _():
        o[...] = jnp.dot(e1[...], w1[...], preferred_element_type=jnp.float32)

    @pl.when((i >= 10) & (i < 20))
    def _():
        o[...] = jnp.dot(e2[...], w2[...], preferred_element_type=jnp.float32)

    @pl.when(i >= 20)
    def _():
        o[...] = jnp.dot(e3[...], w3[...], preferred_element_type=jnp.float32)


def _build_table(emb0, emb1, emb2, emb3, w1, w2, w3):
    nrows = _BLK * sum(_NB)
    return pl.pallas_call(
        _table_body,
        grid=(sum(_NB),),
        in_specs=[
            pl.BlockSpec((_BLK, 128), lambda i: (jnp.clip(i, 0, 4), 0)),
            pl.BlockSpec((_BLK, 64), lambda i: (jnp.clip(i - 5, 0, 4), 0)),
            pl.BlockSpec((_BLK, 32), lambda i: (jnp.clip(i - 10, 0, 9), 0)),
            pl.BlockSpec((_BLK, 16), lambda i: (jnp.clip(i - 20, 0, 29), 0)),
            pl.BlockSpec((64, EMB_DIM), lambda i: (0, 0)),
            pl.BlockSpec((32, EMB_DIM), lambda i: (0, 0)),
            pl.BlockSpec((16, EMB_DIM), lambda i: (0, 0)),
        ],
        out_specs=pl.BlockSpec((_BLK, EMB_DIM), lambda i: (i, 0)),
        out_shape=jax.ShapeDtypeStruct((nrows, EMB_DIM), jnp.float32),
    )(emb0, emb1, emb2, emb3, w1, w2, w3)


def _sc_lookup(x3, table):
    nch = x3.shape[1]
    n_out = _NW * nch * _CH
    mesh = plsc.VectorSubcoreMesh(core_axis_name="c", subcore_axis_name="s")

    @functools.partial(
        pl.kernel,
        out_type=jax.ShapeDtypeStruct((n_out, EMB_DIM), jnp.float32),
        mesh=mesh,
        scratch_types=[
            pltpu.VMEM((nch, _CH), jnp.int32),
            pltpu.VMEM((2, _CH, EMB_DIM), jnp.float32),
            pltpu.SemaphoreType.DMA,
            pltpu.SemaphoreType.DMA,
        ],
    )
    def k(x_hbm, table_hbm, out_hbm, idx_v, rows_v, sem0, sem1):
        wid = lax.axis_index("s") * _NC + lax.axis_index("c")
        cbase = wid * nch
        pltpu.sync_copy(x_hbm.at[wid], idx_v)

        def to_rows(r, carry):
            for c in range(_CH // 16):
                v = idx_v[r, pl.ds(c * 16, 16)]
                shift = (jnp.where(v > 10000, 240, 0)
                         + jnp.where(v > 20000, 240, 0)
                         + jnp.where(v > 40000, 480, 0))
                idx_v[r, pl.ds(c * 16, 16)] = jnp.where(v == 0, 10000,
                                                        v + shift)
            return carry

        lax.fori_loop(0, nch, to_rows, 0)

        def fire(j, buf, sem):
            pltpu.async_copy(table_hbm.at[idx_v.at[j]], buf, sem)

        def wait(j, buf, sem):
            pltpu.make_async_copy(table_hbm.at[idx_v.at[j]], buf, sem).wait()

        def store(j, buf):
            pltpu.sync_copy(buf, out_hbm.at[pl.ds((cbase + j) * _CH, _CH)])

        fire(0, rows_v.at[0], sem0)

        def body(s, carry):
            j0 = 2 * s
            j1 = j0 + 1
            fire(j1, rows_v.at[1], sem1)
            wait(j0, rows_v.at[0], sem0)
            store(j0, rows_v.at[0])
            j2 = jnp.minimum(j0 + 2, nch - 1)
            fire(j2, rows_v.at[0], sem0)
            wait(j1, rows_v.at[1], sem1)
            store(j1, rows_v.at[1])
            return carry

        lax.fori_loop(0, nch // 2, body, 0)
        wait(nch - 1, rows_v.at[0], sem0)

    return k(x3, table)


def kernel(x, emb0, emb1, emb2, emb3, W1, W2, W3):
    table = _build_table(emb0, emb1, emb2, emb3, W1, W2, W3)
    x3 = x.reshape(_NW, -1, _CH).astype(jnp.int32)
    out = _sc_lookup(x3, table)
    return out.reshape(x.shape + (EMB_DIM,))
